# baseline (speedup 1.0000x reference)
import functools

import jax
import jax.numpy as jnp
from jax import lax
from jax.experimental import pallas as pl
from jax.experimental.pallas import tpu as pltpu
from jax.experimental.pallas import tpu_sc as plsc

_N = 20000
_C = 91
_NPAD = 20480
_BPW = 640
_RB = 2000


def _ce_body(x_ref, lab_ref, o_ref):
    i = pl.program_id(0)

    @pl.when(i == 0)
    def _():
        o_ref[...] = jnp.zeros((1, 1), jnp.float32)

    x = x_ref[...]
    lab = lab_ref[...]
    m = jnp.max(x, axis=1, keepdims=True)
    e = jnp.exp(x - m)
    s = jnp.sum(e, axis=1, keepdims=True)
    lse = jnp.log(s) + m
    cols = lax.broadcasted_iota(jnp.int32, x.shape, 1)
    sel = jnp.sum(jnp.where(cols == lab, x, 0.0), axis=1, keepdims=True)
    o_ref[...] += (jnp.sum(lse - sel) * (1.0 / _N)).reshape(1, 1)


def _box_body(box_hbm, lab_hbm, tgt_hbm, sum_hbm, cnt_hbm,
              lab_v, tgt_v, idx_v, pred_v, acc_v, cnt_v, sem):
    c = lax.axis_index("c")
    s = lax.axis_index("s")
    wid = s * 2 + c
    base = wid * _BPW
    nel = _BPW * 4

    pltpu.sync_copy(lab_hbm.at[pl.ds(base, _BPW)], lab_v)
    for k in range(4):
        pltpu.sync_copy(tgt_hbm.at[pl.ds(k * _NPAD + base, _BPW)],
                        tgt_v.at[pl.ds(k * _BPW, _BPW)])

    def idx_body(j, carry):
        lane = lax.iota(jnp.int32, 16)
        lab = lab_v[pl.ds(j * 16, 16)]
        gi = jnp.minimum(base + j * 16 + lane, _N - 1)
        b4 = (gi * _C + lab) * 4
        for k in range(4):
            idx_v[pl.ds(k * _BPW + j * 16, 16)] = b4 + k
        return carry

    lax.fori_loop(0, _BPW // 16, idx_body, 0)

    copies = [
        pltpu.async_copy(
            box_hbm.at[idx_v.at[pl.ds(g * 128, 128)]],
            pred_v.at[pl.ds(g * 128, 128)],
            sem,
        )
        for g in range(nel // 128)
    ]
    for cp in copies:
        cp.wait()

    zero = jnp.zeros((16,), jnp.float32)

    def red_body(j, carry):
        acc, cnt = carry
        lab = lab_v[pl.ds(j * 16, 16)]
        m = jnp.where(lab > 0, 1.0, 0.0)
        cnt = cnt + m
        for k in range(4):
            v = pred_v[pl.ds(k * _BPW + j * 16, 16)]
            t = tgt_v[pl.ds(k * _BPW + j * 16, 16)]
            d = v - t
            ad = jnp.abs(d)
            el = jnp.where(ad < 1.0, 0.5 * d * d, ad - 0.5)
            acc = acc + el * m
        return (acc, cnt)

    acc, cnt = lax.fori_loop(0, _BPW // 16, red_body, (zero, zero))

    acc_v[...] = acc
    cnt_v[...] = cnt
    pltpu.sync_copy(acc_v, sum_hbm.at[wid])
    pltpu.sync_copy(cnt_v, cnt_hbm.at[wid])


def _make_sc_box():
    return functools.partial(
        pl.kernel,
        out_type=(
            jax.ShapeDtypeStruct((32, 16), jnp.float32),
            jax.ShapeDtypeStruct((32, 16), jnp.float32),
        ),
        mesh=plsc.VectorSubcoreMesh(core_axis_name="c", subcore_axis_name="s"),
        scratch_types=[
            pltpu.VMEM((_BPW,), jnp.int32),
            pltpu.VMEM((_BPW * 4,), jnp.float32),
            pltpu.VMEM((_BPW * 4,), jnp.int32),
            pltpu.VMEM((_BPW * 4,), jnp.float32),
            pltpu.VMEM((16,), jnp.float32),
            pltpu.VMEM((16,), jnp.float32),
            pltpu.SemaphoreType.DMA,
        ],
    )(_box_body)


def kernel(class_logits, box_regression, labels, regression_targets):
    labels = labels.astype(jnp.int32)

    nll = pl.pallas_call(
        _ce_body,
        grid=(_N // _RB,),
        in_specs=[
            pl.BlockSpec((_RB, _C), lambda i: (i, 0)),
            pl.BlockSpec((_RB, 1), lambda i: (i, 0)),
        ],
        out_specs=pl.BlockSpec((1, 1), lambda i: (0, 0)),
        out_shape=jax.ShapeDtypeStruct((1, 1), jnp.float32),
    )(class_logits, labels[:, None])

    lab_p = jnp.pad(labels, (0, _NPAD - _N))
    tgt_p = jnp.pad(regression_targets, ((0, _NPAD - _N), (0, 0))).T.reshape(4 * _NPAD)
    box_flat = box_regression.reshape(_N * _C * 4)
    sums, cnts = _make_sc_box()(box_flat, lab_p, tgt_p)

    ce = nll[0, 0]
    box = jnp.sum(sums) / (jnp.sum(cnts) * 4.0)
    return (ce, box)


# speedup vs baseline: 98.2821x; 98.2821x over previous
import jax
import jax.numpy as jnp
from jax import lax
from jax.experimental import pallas as pl

_N = 20000
_C = 91
_IB = 2048
_GRID = (_N + _IB - 1) // _IB


def _loss_body(x_ref, b_ref, lab_ref, t_ref, nll_ref, box_ref, cnt_ref):
    i = pl.program_id(0)

    @pl.when(i == 0)
    def _():
        nll_ref[...] = jnp.zeros((1, 1), jnp.float32)
        box_ref[...] = jnp.zeros((1, 1), jnp.float32)
        cnt_ref[...] = jnp.zeros((1, 1), jnp.float32)

    col = lax.broadcasted_iota(jnp.int32, (1, _IB), 1) + i * _IB
    valid = col < _N

    x = jnp.where(valid, x_ref[...], 0.0)
    lab = lab_ref[...]

    m = jnp.max(x, axis=0, keepdims=True)
    e = jnp.exp(x - m)
    s = jnp.sum(e, axis=0, keepdims=True)
    lse = jnp.log(s) + m

    rows = lax.broadcasted_iota(jnp.int32, (_C, _IB), 0)
    oh = (rows == lab) & valid
    sel = jnp.sum(jnp.where(oh, x, 0.0), axis=0, keepdims=True)
    nll_part = jnp.sum(jnp.where(valid, lse - sel, 0.0))
    nll_ref[...] += (nll_part * (1.0 / _N)).reshape(1, 1)

    pos = (lab > 0) & valid
    bpart = jnp.zeros((), jnp.float32)
    for k in range(4):
        bk = b_ref[:, k, :]
        pred_k = jnp.sum(jnp.where(oh, bk, 0.0), axis=0, keepdims=True)
        d = pred_k - t_ref[k:k + 1, :]
        ad = jnp.abs(d)
        el = jnp.where(ad < 1.0, 0.5 * d * d, ad - 0.5)
        bpart += jnp.sum(jnp.where(pos, el, 0.0))
    box_ref[...] += bpart.reshape(1, 1)
    cnt_ref[...] += jnp.sum(jnp.where(pos, 1.0, 0.0)).reshape(1, 1)


def kernel(class_logits, box_regression, labels, regression_targets):
    labels = labels.astype(jnp.int32)
    lt = class_logits.T
    bt = box_regression.transpose(1, 2, 0)
    tt = regression_targets.T
    lab2 = labels.reshape(1, _N)

    nll, bsum, cnt = pl.pallas_call(
        _loss_body,
        grid=(_GRID,),
        in_specs=[
            pl.BlockSpec((_C, _IB), lambda i: (0, i)),
            pl.BlockSpec((_C, 4, _IB), lambda i: (0, 0, i)),
            pl.BlockSpec((1, _IB), lambda i: (0, i)),
            pl.BlockSpec((4, _IB), lambda i: (0, i)),
        ],
        out_specs=[
            pl.BlockSpec((1, 1), lambda i: (0, 0)),
            pl.BlockSpec((1, 1), lambda i: (0, 0)),
            pl.BlockSpec((1, 1), lambda i: (0, 0)),
        ],
        out_shape=[
            jax.ShapeDtypeStruct((1, 1), jnp.float32),
            jax.ShapeDtypeStruct((1, 1), jnp.float32),
            jax.ShapeDtypeStruct((1, 1), jnp.float32),
        ],
    )(lt, bt, lab2, tt)

    ce = nll[0, 0]
    box = bsum[0, 0] / (cnt[0, 0] * 4.0)
    return (ce, box)


# speedup vs baseline: 98.8405x; 1.0057x over previous
import jax
import jax.numpy as jnp
from jax import lax
from jax.experimental import pallas as pl

_N = 20000
_C = 91
_IB = 2048
_GRID = (_N + _IB - 1) // _IB


def _loss_body(x_ref, b_ref, lab_ref, t_ref, nll_ref, box_ref, cnt_ref):
    i = pl.program_id(0)

    @pl.when(i == 0)
    def _():
        nll_ref[...] = jnp.zeros((1, 1), jnp.float32)
        box_ref[...] = jnp.zeros((1, 1), jnp.float32)
        cnt_ref[...] = jnp.zeros((1, 1), jnp.float32)

    col = lax.broadcasted_iota(jnp.int32, (1, _IB), 1) + i * _IB
    valid = col < _N

    x = x_ref[...]
    lab = lab_ref[...]

    m = jnp.max(x, axis=0, keepdims=True)
    e = jnp.exp(x - m)
    s = jnp.sum(e, axis=0, keepdims=True)
    lse = jnp.log(s) + m

    rows = lax.broadcasted_iota(jnp.int32, (_C, _IB), 0)
    oh = rows == lab
    sel = jnp.sum(jnp.where(oh, x, 0.0), axis=0, keepdims=True)
    nll_part = jnp.sum(jnp.where(valid, lse - sel, 0.0))
    nll_ref[...] += (nll_part * (1.0 / _N)).reshape(1, 1)

    pos = (lab > 0) & valid
    bpart = jnp.zeros((), jnp.float32)
    for k in range(4):
        bk = b_ref[:, k, :]
        pred_k = jnp.sum(jnp.where(oh, bk, 0.0), axis=0, keepdims=True)
        d = pred_k - t_ref[k:k + 1, :]
        ad = jnp.abs(d)
        el = jnp.where(ad < 1.0, 0.5 * d * d, ad - 0.5)
        bpart += jnp.sum(jnp.where(pos, el, 0.0))
    box_ref[...] += bpart.reshape(1, 1)
    cnt_ref[...] += jnp.sum(jnp.where(pos, 1.0, 0.0)).reshape(1, 1)


def kernel(class_logits, box_regression, labels, regression_targets):
    labels = labels.astype(jnp.int32)
    lt = class_logits.T
    bt = box_regression.transpose(1, 2, 0)
    tt = regression_targets.T
    lab2 = labels.reshape(1, _N)

    nll, bsum, cnt = pl.pallas_call(
        _loss_body,
        grid=(_GRID,),
        in_specs=[
            pl.BlockSpec((_C, _IB), lambda i: (0, i)),
            pl.BlockSpec((_C, 4, _IB), lambda i: (0, 0, i)),
            pl.BlockSpec((1, _IB), lambda i: (0, i)),
            pl.BlockSpec((4, _IB), lambda i: (0, i)),
        ],
        out_specs=[
            pl.BlockSpec((1, 1), lambda i: (0, 0)),
            pl.BlockSpec((1, 1), lambda i: (0, 0)),
            pl.BlockSpec((1, 1), lambda i: (0, 0)),
        ],
        out_shape=[
            jax.ShapeDtypeStruct((1, 1), jnp.float32),
            jax.ShapeDtypeStruct((1, 1), jnp.float32),
            jax.ShapeDtypeStruct((1, 1), jnp.float32),
        ],
    )(lt, bt, lab2, tt)

    ce = nll[0, 0]
    box = bsum[0, 0] / (cnt[0, 0] * 4.0)
    return (ce, box)


# speedup vs baseline: 101.5208x; 1.0271x over previous
import jax
import jax.numpy as jnp
from jax import lax
from jax.experimental import pallas as pl

_N = 20000
_C = 91
_IB = 4096
_GRID = (_N + _IB - 1) // _IB


def _loss_body(x_ref, b_ref, lab_ref, t_ref, nll_ref, box_ref, cnt_ref):
    i = pl.program_id(0)

    @pl.when(i == 0)
    def _():
        nll_ref[...] = jnp.zeros((1, 1), jnp.float32)
        box_ref[...] = jnp.zeros((1, 1), jnp.float32)
        cnt_ref[...] = jnp.zeros((1, 1), jnp.float32)

    col = lax.broadcasted_iota(jnp.int32, (1, _IB), 1) + i * _IB
    valid = col < _N

    x = x_ref[...]
    lab = lab_ref[...]

    m = jnp.max(x, axis=0, keepdims=True)
    e = jnp.exp(x - m)
    s = jnp.sum(e, axis=0, keepdims=True)
    lse = jnp.log(s) + m

    rows = lax.broadcasted_iota(jnp.int32, (_C, _IB), 0)
    oh = rows == lab
    sel = jnp.sum(jnp.where(oh, x, 0.0), axis=0, keepdims=True)
    nll_part = jnp.sum(jnp.where(valid, lse - sel, 0.0))
    nll_ref[...] += (nll_part * (1.0 / _N)).reshape(1, 1)

    pos = (lab > 0) & valid
    bpart = jnp.zeros((), jnp.float32)
    for k in range(4):
        bk = b_ref[:, k, :]
        pred_k = jnp.sum(jnp.where(oh, bk, 0.0), axis=0, keepdims=True)
        d = pred_k - t_ref[k:k + 1, :]
        ad = jnp.abs(d)
        el = jnp.where(ad < 1.0, 0.5 * d * d, ad - 0.5)
        bpart += jnp.sum(jnp.where(pos, el, 0.0))
    box_ref[...] += bpart.reshape(1, 1)
    cnt_ref[...] += jnp.sum(jnp.where(pos, 1.0, 0.0)).reshape(1, 1)


def kernel(class_logits, box_regression, labels, regression_targets):
    labels = labels.astype(jnp.int32)
    lt = class_logits.T
    bt = box_regression.transpose(1, 2, 0)
    tt = regression_targets.T
    lab2 = labels.reshape(1, _N)

    nll, bsum, cnt = pl.pallas_call(
        _loss_body,
        grid=(_GRID,),
        in_specs=[
            pl.BlockSpec((_C, _IB), lambda i: (0, i)),
            pl.BlockSpec((_C, 4, _IB), lambda i: (0, 0, i)),
            pl.BlockSpec((1, _IB), lambda i: (0, i)),
            pl.BlockSpec((4, _IB), lambda i: (0, i)),
        ],
        out_specs=[
            pl.BlockSpec((1, 1), lambda i: (0, 0)),
            pl.BlockSpec((1, 1), lambda i: (0, 0)),
            pl.BlockSpec((1, 1), lambda i: (0, 0)),
        ],
        out_shape=[
            jax.ShapeDtypeStruct((1, 1), jnp.float32),
            jax.ShapeDtypeStruct((1, 1), jnp.float32),
            jax.ShapeDtypeStruct((1, 1), jnp.float32),
        ],
    )(lt, bt, lab2, tt)

    ce = nll[0, 0]
    box = bsum[0, 0] / (cnt[0, 0] * 4.0)
    return (ce, box)


# speedup vs baseline: 102.0270x; 1.0050x over previous
import jax
import jax.numpy as jnp
from jax import lax
from jax.experimental import pallas as pl

_N = 20000
_C = 91
_IB = 5120
_GRID = (_N + _IB - 1) // _IB


def _loss_body(x_ref, b_ref, lab_ref, t_ref, nll_ref, box_ref, cnt_ref):
    i = pl.program_id(0)

    @pl.when(i == 0)
    def _():
        nll_ref[...] = jnp.zeros((1, 1), jnp.float32)
        box_ref[...] = jnp.zeros((1, 1), jnp.float32)
        cnt_ref[...] = jnp.zeros((1, 1), jnp.float32)

    col = lax.broadcasted_iota(jnp.int32, (1, _IB), 1) + i * _IB
    valid = col < _N

    x = x_ref[...]
    lab = lab_ref[...]

    m = jnp.max(x, axis=0, keepdims=True)
    e = jnp.exp(x - m)
    s = jnp.sum(e, axis=0, keepdims=True)
    lse = jnp.log(s) + m

    rows = lax.broadcasted_iota(jnp.int32, (_C, _IB), 0)
    oh = rows == lab
    sel = jnp.sum(jnp.where(oh, x, 0.0), axis=0, keepdims=True)
    nll_part = jnp.sum(jnp.where(valid, lse - sel, 0.0))
    nll_ref[...] += (nll_part * (1.0 / _N)).reshape(1, 1)

    pos = (lab > 0) & valid
    bpart = jnp.zeros((), jnp.float32)
    for k in range(4):
        bk = b_ref[:, k, :]
        pred_k = jnp.sum(jnp.where(oh, bk, 0.0), axis=0, keepdims=True)
        d = pred_k - t_ref[k:k + 1, :]
        ad = jnp.abs(d)
        el = jnp.where(ad < 1.0, 0.5 * d * d, ad - 0.5)
        bpart += jnp.sum(jnp.where(pos, el, 0.0))
    box_ref[...] += bpart.reshape(1, 1)
    cnt_ref[...] += jnp.sum(jnp.where(pos, 1.0, 0.0)).reshape(1, 1)


def kernel(class_logits, box_regression, labels, regression_targets):
    labels = labels.astype(jnp.int32)
    lt = class_logits.T
    bt = box_regression.transpose(1, 2, 0)
    tt = regression_targets.T
    lab2 = labels.reshape(1, _N)

    nll, bsum, cnt = pl.pallas_call(
        _loss_body,
        grid=(_GRID,),
        in_specs=[
            pl.BlockSpec((_C, _IB), lambda i: (0, i)),
            pl.BlockSpec((_C, 4, _IB), lambda i: (0, 0, i)),
            pl.BlockSpec((1, _IB), lambda i: (0, i)),
            pl.BlockSpec((4, _IB), lambda i: (0, i)),
        ],
        out_specs=[
            pl.BlockSpec((1, 1), lambda i: (0, 0)),
            pl.BlockSpec((1, 1), lambda i: (0, 0)),
            pl.BlockSpec((1, 1), lambda i: (0, 0)),
        ],
        out_shape=[
            jax.ShapeDtypeStruct((1, 1), jnp.float32),
            jax.ShapeDtypeStruct((1, 1), jnp.float32),
            jax.ShapeDtypeStruct((1, 1), jnp.float32),
        ],
    )(lt, bt, lab2, tt)

    ce = nll[0, 0]
    box = bsum[0, 0] / (cnt[0, 0] * 4.0)
    return (ce, box)
